# baseline (speedup 1.0000x reference)
import functools

import jax
import jax.numpy as jnp
from jax import lax
from jax.experimental import pallas as pl
from jax.experimental.pallas import tpu as pltpu
from jax.experimental.pallas import tpu_sc as plsc

D = 64
B = 4096
H = 200
L = 16
NC, NS = 2, 16
NW = NC * NS
HT = H // 8
BT = B // 128
NTASK = H


def _rsqrt(v):
    i = lax.bitcast_convert_type(v, jnp.int32)
    i = jnp.int32(0x5F3759DF) - lax.shift_right_logical(i, 1)
    y = lax.bitcast_convert_type(i, jnp.float32)
    for _ in range(3):
        y = y * (1.5 - 0.5 * v * y * y)
    return y


@functools.partial(
    pl.kernel,
    mesh=plsc.VectorSubcoreMesh(core_axis_name="c", subcore_axis_name="s"),
    out_type=jax.ShapeDtypeStruct((H, D // 8, BT, 8, 128), jnp.float32),
    scratch_types=[
        pltpu.VMEM((HT, 8, 128), jnp.int32),
        pltpu.VMEM((128, D), jnp.float32),
        pltpu.VMEM((128, D), jnp.float32),
        pltpu.VMEM((D // 8, 8, 128), jnp.float32),
        pltpu.VMEM((D,), jnp.float32),
        pltpu.VMEM((D,), jnp.float32),
        pltpu.SemaphoreType.DMA,
        pltpu.SemaphoreType.DMA,
    ],
    compiler_params=pltpu.CompilerParams(
        needs_layout_passes=False, use_tc_tiling_on_sc=False),
)
def _sc_embed_ln(x4_hbm, table_hbm, gamma_hbm, beta_hbm, out_hbm,
                 idxs, rows_a, rows_b, tr_v, gamma_v, beta_v, sem_a, sem_b):
    w = lax.axis_index("s") * NC + lax.axis_index("c")

    for ht in range(HT):
        pltpu.sync_copy(x4_hbm.at[ht, w], idxs.at[ht])
    pltpu.sync_copy(gamma_hbm, gamma_v)
    pltpu.sync_copy(beta_hbm, beta_v)
    g = [gamma_v[pl.ds(k * L, L)] for k in range(D // L)]
    b = [beta_v[pl.ds(k * L, L)] for k in range(D // L)]

    iota = lax.iota(jnp.int32, L)
    ci = lax.rem(iota, 8)
    ti = [lax.div(iota + k * L, 8) for k in range(D // L)]

    bufs = ((rows_a, sem_a), (rows_b, sem_b))

    def fire_gather(t, par):
        rows, sem = bufs[par]
        pltpu.async_copy(table_hbm.at[idxs.at[lax.div(t, 8), lax.rem(t, 8)]],
                         rows, sem)

    def wait_gather(t, par):
        rows, sem = bufs[par]
        pltpu.make_async_copy(table_hbm.at[idxs.at[lax.div(t, 8), lax.rem(t, 8)]],
                              rows, sem).wait()

    def normalize_transpose(rows):
        def row_body(r, carry):
            v = [rows[r, pl.ds(k * L, L)] for k in range(D // L)]
            s = (v[0] + v[1]) + (v[2] + v[3])
            sq = (v[0] * v[0] + v[1] * v[1]) + (v[2] * v[2] + v[3] * v[3])
            mean = jnp.sum(s) * (1.0 / D)
            var = jnp.sum(sq) * (1.0 / D) - mean * mean
            rstd = _rsqrt(var + 1e-5)
            rr = jnp.full((L,), r, jnp.int32)
            for k in range(D // L):
                o = (v[k] - mean) * rstd * g[k] + b[k]
                plsc.store_scatter(tr_v, [ti[k], ci, rr], o)
            return carry

        lax.fori_loop(0, 128, row_body, 0, unroll=4)

    fire_gather(0, 0)

    def pair_body(i, carry):
        for par in range(2):
            t = i * 2 + par
            wait_gather(t, par)

            @pl.when(t + 1 < NTASK)
            def _():
                fire_gather(t + 1, 1 - par)

            rows, _ = bufs[par]
            normalize_transpose(rows)
            for k in range(D // 8):
                pltpu.sync_copy(tr_v.at[k], out_hbm.at[t, k, w])
        return carry

    lax.fori_loop(0, NTASK // 2, pair_body, 0)


def kernel(x, table, gamma, beta):
    x4 = x.astype(jnp.int32).T.reshape(HT, 8, BT, 128).transpose(0, 2, 1, 3)
    out5 = _sc_embed_ln(x4, table, gamma, beta)
    return out5.transpose(2, 4, 0, 1, 3).reshape(B, H, D)
